# Initial kernel scaffold; baseline (speedup 1.0000x reference)
import jax
import jax.numpy as jnp
from jax.experimental import pallas as pl


def kernel(query, key, value):
    raise NotImplementedError("write your pallas kernel here")


# speedup vs baseline: 14.6988x; 14.6988x over previous
import functools

import jax
import jax.numpy as jnp
from jax.experimental import pallas as pl
from jax.experimental.pallas import tpu as pltpu

TOPK = 64
TQ = 256
SQ = 2048
SK = 2048
D = 1024
INT_MIN = -2147483648


def _sortable(x):
    i = jax.lax.bitcast_convert_type(x, jnp.int32)
    return jnp.where(i < 0, i ^ jnp.int32(0x7FFFFFFF), i)


def _phase1_body(q_ref, k_ref, masked_ref, colsq_ref):
    qt = pl.program_id(1)
    s = jax.lax.dot_general(
        q_ref[0],
        k_ref[0],
        dimension_numbers=(((1,), (1,)), ((), ())),
        preferred_element_type=jnp.float32,
    )
    ss = _sortable(s)

    cnt_pos = jnp.sum((ss >= 0).astype(jnp.int32), axis=1, keepdims=True)
    t = jnp.where(cnt_pos >= TOPK, jnp.int32(0), jnp.int32(INT_MIN))
    for b in range(30, -1, -1):
        trial = t | jnp.int32(1 << b)
        cnt = jnp.sum((ss >= trial).astype(jnp.int32), axis=1, keepdims=True)
        t = jnp.where(cnt >= TOPK, trial, t)

    masked = jnp.where(ss >= t, s, 0.0)
    masked_ref[0] = masked.astype(jnp.bfloat16)

    @pl.when(qt == 0)
    def _init():
        colsq_ref[...] = jnp.zeros_like(colsq_ref)

    colsq_ref[...] += jnp.broadcast_to(
        jnp.sum(masked * masked, axis=0)[None, None, :], colsq_ref.shape
    )


def _phase2_body(masked_ref, colsq_ref, v_ref, out_ref):
    colsq = colsq_ref[0, 0]
    scale = jax.lax.rsqrt(jnp.maximum(colsq, 1e-24))
    scale = jnp.where(colsq == 0.0, 0.0, scale)
    scale = jnp.minimum(scale, 1e12)
    att = (masked_ref[0].astype(jnp.float32) * scale[None, :]).astype(jnp.bfloat16)
    out_ref[0] = jax.lax.dot_general(
        att,
        v_ref[0].astype(jnp.bfloat16),
        dimension_numbers=(((1,), (0,)), ((), ())),
        preferred_element_type=jnp.float32,
    )


@jax.jit
def kernel(query, key, value):
    B = query.shape[0]
    grid = (B, SQ // TQ)

    masked, colsq = pl.pallas_call(
        _phase1_body,
        grid=grid,
        in_specs=[
            pl.BlockSpec((1, TQ, D), lambda b, q: (b, q, 0)),
            pl.BlockSpec((1, SK, D), lambda b, q: (b, 0, 0)),
        ],
        out_specs=[
            pl.BlockSpec((1, TQ, SK), lambda b, q: (b, q, 0)),
            pl.BlockSpec((1, 8, SK), lambda b, q: (b, 0, 0)),
        ],
        out_shape=[
            jax.ShapeDtypeStruct((B, SQ, SK), jnp.bfloat16),
            jax.ShapeDtypeStruct((B, 8, SK), jnp.float32),
        ],
        compiler_params=pltpu.CompilerParams(
            dimension_semantics=("arbitrary", "arbitrary"),
        ),
    )(query, key)

    out = pl.pallas_call(
        _phase2_body,
        grid=grid,
        in_specs=[
            pl.BlockSpec((1, TQ, SK), lambda b, q: (b, q, 0)),
            pl.BlockSpec((1, 8, SK), lambda b, q: (b, 0, 0)),
            pl.BlockSpec((1, SK, D), lambda b, q: (b, 0, 0)),
        ],
        out_specs=pl.BlockSpec((1, TQ, D), lambda b, q: (b, q, 0)),
        out_shape=jax.ShapeDtypeStruct((B, SQ, D), jnp.float32),
        compiler_params=pltpu.CompilerParams(
            dimension_semantics=("arbitrary", "arbitrary"),
        ),
    )(masked, colsq, value)
    return out
